# initial kernel scaffold (unmeasured)
import jax
import jax.numpy as jnp
from jax import lax
from jax.experimental import pallas as pl
from jax.experimental.pallas import tpu as pltpu

N_DEV = 32


def kernel(x, w_mat):
    m_per, k = x.shape
    n = w_mat.shape[1]
    n_per = n // N_DEV

    def body(x_ref, w_ref, out_ref, y_ref, recv_ref, send_sems, recv_sems):
        me = lax.axis_index("i")

        barrier_sem = pltpu.get_barrier_semaphore()
        for off in range(1, N_DEV):
            peer = lax.rem(me + off, N_DEV)
            pl.semaphore_signal(
                barrier_sem, inc=1,
                device_id=(peer,), device_id_type=pl.DeviceIdType.MESH,
            )
        pl.semaphore_wait(barrier_sem, N_DEV - 1)

        y_ref[:, :] = jnp.maximum(
            jnp.dot(x_ref[:, :], w_ref[:, :],
                    preferred_element_type=jnp.float32),
            0.0,
        )

        out_ref[pl.ds(me * m_per, m_per), :] = y_ref[:, pl.ds(me * n_per, n_per)]

        sends = []
        for off in range(1, N_DEV):
            dst = lax.rem(me + off, N_DEV)
            rdma = pltpu.make_async_remote_copy(
                src_ref=y_ref.at[:, pl.ds(dst * n_per, n_per)],
                dst_ref=recv_ref.at[me],
                send_sem=send_sems.at[dst],
                recv_sem=recv_sems.at[me],
                device_id=(dst,),
                device_id_type=pl.DeviceIdType.MESH,
            )
            rdma.start()
            sends.append(rdma)

        for off in range(1, N_DEV):
            src = lax.rem(me + off, N_DEV)
            recv = pltpu.make_async_remote_copy(
                src_ref=y_ref.at[:, pl.ds(0, n_per)],
                dst_ref=recv_ref.at[src],
                send_sem=send_sems.at[src],
                recv_sem=recv_sems.at[src],
                device_id=(src,),
                device_id_type=pl.DeviceIdType.MESH,
            )
            recv.wait_recv()
            out_ref[pl.ds(src * m_per, m_per), :] = recv_ref[src]

        for rdma in sends:
            rdma.wait_send()

    return pl.pallas_call(
        body,
        out_shape=jax.ShapeDtypeStruct((n, n_per), jnp.float32),
        in_specs=[
            pl.BlockSpec(memory_space=pltpu.VMEM),
            pl.BlockSpec(memory_space=pltpu.VMEM),
        ],
        out_specs=pl.BlockSpec(memory_space=pltpu.VMEM),
        scratch_shapes=[
            pltpu.VMEM((m_per, n), jnp.float32),
            pltpu.VMEM((N_DEV, m_per, n_per), jnp.float32),
            pltpu.SemaphoreType.DMA((N_DEV,)),
            pltpu.SemaphoreType.DMA((N_DEV,)),
        ],
        compiler_params=pltpu.CompilerParams(collective_id=0),
    )(x, w_mat)


# baseline (device time: 20240 ns/iter reference)
import jax
import jax.numpy as jnp
from jax import lax
from jax.experimental import pallas as pl
from jax.experimental.pallas import tpu as pltpu

N_DEV = 32


def kernel(x, w_mat):
    m_per, k = x.shape
    n = w_mat.shape[1]
    n_per = n // N_DEV

    def body(x_ref, w_ref, out_ref, stage_ref, recv_ref, send_sems, recv_sems):
        me = lax.axis_index("i")

        barrier_sem = pltpu.get_barrier_semaphore()
        for off in range(1, N_DEV):
            peer = lax.rem(me + off, N_DEV)
            pl.semaphore_signal(
                barrier_sem, inc=1,
                device_id=(peer,), device_id_type=pl.DeviceIdType.MESH,
            )
        pl.semaphore_wait(barrier_sem, N_DEV - 1)

        y = jnp.maximum(
            jnp.dot(x_ref[:, :], w_ref[:, :],
                    preferred_element_type=jnp.float32),
            0.0,
        )
        for j in range(N_DEV):
            stage_ref[j, :, :] = y[:, j * n_per:(j + 1) * n_per]

        out_ref[pl.ds(me * m_per, m_per), :] = stage_ref[me]

        sends = []
        for off in range(1, N_DEV):
            dst = lax.rem(me + off, N_DEV)
            rdma = pltpu.make_async_remote_copy(
                src_ref=stage_ref.at[dst],
                dst_ref=recv_ref.at[me],
                send_sem=send_sems.at[dst],
                recv_sem=recv_sems.at[me],
                device_id=(dst,),
                device_id_type=pl.DeviceIdType.MESH,
            )
            rdma.start()
            sends.append(rdma)

        for off in range(1, N_DEV):
            src = lax.rem(me + off, N_DEV)
            recv = pltpu.make_async_remote_copy(
                src_ref=stage_ref.at[src],
                dst_ref=recv_ref.at[src],
                send_sem=send_sems.at[src],
                recv_sem=recv_sems.at[src],
                device_id=(src,),
                device_id_type=pl.DeviceIdType.MESH,
            )
            recv.wait_recv()
            out_ref[pl.ds(src * m_per, m_per), :] = recv_ref[src]

        for rdma in sends:
            rdma.wait_send()

    return pl.pallas_call(
        body,
        out_shape=jax.ShapeDtypeStruct((n, n_per), jnp.float32),
        in_specs=[
            pl.BlockSpec(memory_space=pltpu.VMEM),
            pl.BlockSpec(memory_space=pltpu.VMEM),
        ],
        out_specs=pl.BlockSpec(memory_space=pltpu.VMEM),
        scratch_shapes=[
            pltpu.VMEM((N_DEV, m_per, n_per), jnp.float32),
            pltpu.VMEM((N_DEV, m_per, n_per), jnp.float32),
            pltpu.SemaphoreType.DMA((N_DEV,)),
            pltpu.SemaphoreType.DMA((N_DEV,)),
        ],
        compiler_params=pltpu.CompilerParams(collective_id=0),
    )(x, w_mat)


# device time: 14651 ns/iter; 1.3815x vs baseline; 1.3815x over previous
import jax
import jax.numpy as jnp
from jax import lax
from jax.experimental import pallas as pl
from jax.experimental.pallas import tpu as pltpu

N_DEV = 32
N_GROUPS = 4
PER_GROUP = N_DEV // N_GROUPS


def kernel(x, w_mat):
    m_per, k = x.shape
    n = w_mat.shape[1]
    n_per = n // N_DEV
    cols_per_group = n // N_GROUPS

    def body(x_ref, w_ref, out_ref, stage_ref, recv_ref, send_sems, recv_sems):
        me = lax.axis_index("i")

        barrier_sem = pltpu.get_barrier_semaphore()
        for off in range(1, N_DEV):
            peer = lax.rem(me + off, N_DEV)
            pl.semaphore_signal(
                barrier_sem, inc=1,
                device_id=(peer,), device_id_type=pl.DeviceIdType.MESH,
            )

        sends = []
        for g in range(N_GROUPS):
            yg = jnp.maximum(
                jnp.dot(x_ref[:, :],
                        w_ref[:, g * cols_per_group:(g + 1) * cols_per_group],
                        preferred_element_type=jnp.float32),
                0.0,
            )
            for i in range(PER_GROUP):
                d = g * PER_GROUP + i
                blk = yg[:, i * n_per:(i + 1) * n_per]
                stage_ref[d, :, :] = jnp.concatenate(
                    [blk[8 * q:8 * (q + 1), :] for q in range(4)], axis=1
                )

            if g == 0:
                pl.semaphore_wait(barrier_sem, N_DEV - 1)

            for i in range(PER_GROUP):
                dst = g * PER_GROUP + lax.rem(
                    jnp.int32(i) + me, jnp.int32(PER_GROUP)
                )
                rdma = pltpu.make_async_remote_copy(
                    src_ref=stage_ref.at[dst],
                    dst_ref=recv_ref.at[me],
                    send_sem=send_sems.at[dst],
                    recv_sem=recv_sems.at[me],
                    device_id=(dst,),
                    device_id_type=pl.DeviceIdType.MESH,
                )
                rdma.start()
                sends.append(rdma)

        for s in range(N_DEV):
            recv = pltpu.make_async_remote_copy(
                src_ref=stage_ref.at[s],
                dst_ref=recv_ref.at[s],
                send_sem=send_sems.at[s],
                recv_sem=recv_sems.at[s],
                device_id=(jnp.int32(s),),
                device_id_type=pl.DeviceIdType.MESH,
            )
            recv.wait_recv()
            for q in range(4):
                out_ref[pl.ds(s * m_per + 8 * q, 8), :] = (
                    recv_ref[s, :, 32 * q:32 * (q + 1)]
                )

        for rdma in sends:
            rdma.wait_send()

    return pl.pallas_call(
        body,
        out_shape=jax.ShapeDtypeStruct((n, n_per), jnp.float32),
        in_specs=[
            pl.BlockSpec(memory_space=pltpu.VMEM),
            pl.BlockSpec(memory_space=pltpu.VMEM),
        ],
        out_specs=pl.BlockSpec(memory_space=pltpu.VMEM),
        scratch_shapes=[
            pltpu.VMEM((N_DEV, 8, 128), jnp.float32),
            pltpu.VMEM((N_DEV, 8, 128), jnp.float32),
            pltpu.SemaphoreType.DMA((N_DEV,)),
            pltpu.SemaphoreType.DMA((N_DEV,)),
        ],
        compiler_params=pltpu.CompilerParams(collective_id=0),
    )(x, w_mat)
